# baseline (speedup 1.0000x reference)
import functools

import jax
import jax.numpy as jnp
from jax import lax
from jax.experimental import pallas as pl
from jax.experimental.pallas import tpu as pltpu
from jax.experimental.pallas import tpu_sc as plsc

_CHUNK = 128
_NBUF = 8
_LAG = 4


def _make_gather(num_rows: int, vocab: int, dim: int):
    info = plsc.get_sparse_core_info()
    nc, ns = info.num_cores, info.num_subcores
    nw = nc * ns
    assert num_rows % (nw * _CHUNK * _NBUF) == 0
    per_w = num_rows // nw
    n_chunks = per_w // _CHUNK
    n_outer = n_chunks // _NBUF
    half_groups = n_outer // 2

    mesh = plsc.VectorSubcoreMesh(core_axis_name="c", subcore_axis_name="s")

    @functools.partial(
        pl.kernel,
        mesh=mesh,
        compiler_params=pltpu.CompilerParams(use_tc_tiling_on_sc=False),
        out_type=jax.ShapeDtypeStruct((num_rows, dim), jnp.bfloat16),
        scratch_types=[
            pltpu.VMEM((n_chunks, _CHUNK), jnp.int32),
            pltpu.VMEM((_NBUF, _CHUNK, dim), jnp.bfloat16),
            pltpu.SemaphoreType.DMA((_NBUF,)),
            pltpu.SemaphoreType.DMA((_NBUF,)),
            pltpu.SemaphoreType.DMA,
        ],
    )
    def emb(idx_hbm, tab_hbm, out_hbm, idx_v, rows_v, gsem, wsem, isem):
        wid = lax.axis_index("s") * nc + lax.axis_index("c")
        base = wid * per_w
        half = half_groups * _NBUF

        def gather(c, b):
            return pltpu.make_async_copy(
                tab_hbm.at[idx_v.at[c]], rows_v.at[b], gsem.at[b]
            )

        def writeback(c, b):
            return pltpu.make_async_copy(
                rows_v.at[b], out_hbm.at[pl.ds(base + c * _CHUNK, _CHUNK)],
                wsem.at[b],
            )

        idx_half1 = pltpu.make_async_copy(
            idx_hbm.at[wid, pl.ds(0, half)], idx_v.at[pl.ds(0, half)], isem
        )
        idx_half2 = pltpu.make_async_copy(
            idx_hbm.at[wid, pl.ds(half, n_chunks - half)],
            idx_v.at[pl.ds(half, n_chunks - half)],
            isem,
        )
        idx_half1.start()
        idx_half2.start()
        idx_half1.wait()

        def outer(g, carry):
            @pl.when(g == half_groups)
            def _wait_half2():
                idx_half2.wait()

            for b in range(_NBUF):
                c = g * _NBUF + b

                @pl.when(c >= _NBUF)
                def _wait_wb():
                    pltpu.make_async_copy(
                        rows_v.at[b], out_hbm.at[pl.ds(base, _CHUNK)],
                        wsem.at[b],
                    ).wait()

                gather(c, b).start()

                bd = (b + _NBUF - _LAG) % _NBUF
                cd = c - _LAG

                @pl.when(cd >= 0)
                def _drain():
                    pltpu.make_async_copy(
                        tab_hbm.at[idx_v.at[0]], rows_v.at[bd], gsem.at[bd]
                    ).wait()
                    writeback(cd, bd).start()
            return carry

        lax.fori_loop(0, n_outer, outer, 0)

        for t in range(_LAG):
            c = n_chunks - _LAG + t
            b = c % _NBUF
            pltpu.make_async_copy(
                tab_hbm.at[idx_v.at[0]], rows_v.at[b], gsem.at[b]
            ).wait()
            writeback(c, b).start()
        for b in range(_NBUF):
            pltpu.make_async_copy(
                rows_v.at[b], out_hbm.at[pl.ds(base, _CHUNK)], wsem.at[b]
            ).wait()

    return emb


def kernel(token_ids, embedding_matrix):
    b, h = token_ids.shape
    v, d = embedding_matrix.shape
    info = plsc.get_sparse_core_info()
    nw = info.num_cores * info.num_subcores
    flat = token_ids.reshape(nw, (b * h) // (nw * _CHUNK), _CHUNK).astype(jnp.int32)
    tab_bf16 = embedding_matrix.astype(jnp.bfloat16)
    emb = _make_gather(b * h, v, d)
    out = emb(flat, tab_bf16)
    return out.astype(jnp.float32).reshape(b, h, d)


# speedup vs baseline: 1.5570x; 1.5570x over previous
import functools

import jax
import jax.numpy as jnp
from jax import lax
from jax.experimental import pallas as pl
from jax.experimental.pallas import tpu as pltpu
from jax.experimental.pallas import tpu_sc as plsc

_CHUNK = 256
_NBUF = 5
_LAG = 2


def _make_gather(num_rows: int, vocab: int, dim: int):
    info = plsc.get_sparse_core_info()
    nc, ns = info.num_cores, info.num_subcores
    nw = nc * ns
    assert num_rows % (nw * _CHUNK * _NBUF) == 0
    per_w = num_rows // nw
    n_chunks = per_w // _CHUNK
    n_outer = n_chunks // _NBUF
    half_groups = n_outer // 2

    mesh = plsc.VectorSubcoreMesh(core_axis_name="c", subcore_axis_name="s")

    @functools.partial(
        pl.kernel,
        mesh=mesh,
        compiler_params=pltpu.CompilerParams(use_tc_tiling_on_sc=False),
        out_type=jax.ShapeDtypeStruct((num_rows, dim), jnp.float32),
        scratch_types=[
            pltpu.VMEM((n_chunks, _CHUNK), jnp.int32),
            pltpu.VMEM((_NBUF, _CHUNK, dim), jnp.float32),
            pltpu.SemaphoreType.DMA((_NBUF,)),
            pltpu.SemaphoreType.DMA((_NBUF,)),
            pltpu.SemaphoreType.DMA,
        ],
    )
    def emb(idx_hbm, tab_hbm, out_hbm, idx_v, rows_v, gsem, wsem, isem):
        wid = lax.axis_index("s") * nc + lax.axis_index("c")
        base = wid * per_w
        half = half_groups * _NBUF

        def gather(c, b):
            return pltpu.make_async_copy(
                tab_hbm.at[idx_v.at[c]], rows_v.at[b], gsem.at[b]
            )

        def writeback(c, b):
            return pltpu.make_async_copy(
                rows_v.at[b], out_hbm.at[pl.ds(base + c * _CHUNK, _CHUNK)],
                wsem.at[b],
            )

        idx_half1 = pltpu.make_async_copy(
            idx_hbm.at[wid, pl.ds(0, half)], idx_v.at[pl.ds(0, half)], isem
        )
        idx_half2 = pltpu.make_async_copy(
            idx_hbm.at[wid, pl.ds(half, n_chunks - half)],
            idx_v.at[pl.ds(half, n_chunks - half)],
            isem,
        )
        idx_half1.start()
        idx_half2.start()
        idx_half1.wait()

        def outer(g, carry):
            @pl.when(g == half_groups)
            def _wait_half2():
                idx_half2.wait()

            for b in range(_NBUF):
                c = g * _NBUF + b

                @pl.when(c >= _NBUF)
                def _wait_wb():
                    pltpu.make_async_copy(
                        rows_v.at[b], out_hbm.at[pl.ds(base, _CHUNK)],
                        wsem.at[b],
                    ).wait()

                gather(c, b).start()

                bd = (b + _NBUF - _LAG) % _NBUF
                cd = c - _LAG

                @pl.when(cd >= 0)
                def _drain():
                    pltpu.make_async_copy(
                        tab_hbm.at[idx_v.at[0]], rows_v.at[bd], gsem.at[bd]
                    ).wait()
                    writeback(cd, bd).start()
            return carry

        lax.fori_loop(0, n_outer, outer, 0)

        for t in range(_LAG):
            c = n_chunks - _LAG + t
            b = c % _NBUF
            pltpu.make_async_copy(
                tab_hbm.at[idx_v.at[0]], rows_v.at[b], gsem.at[b]
            ).wait()
            writeback(c, b).start()
        for b in range(_NBUF):
            pltpu.make_async_copy(
                rows_v.at[b], out_hbm.at[pl.ds(base, _CHUNK)], wsem.at[b]
            ).wait()

    return emb


def kernel(token_ids, embedding_matrix):
    b, h = token_ids.shape
    v, d = embedding_matrix.shape
    info = plsc.get_sparse_core_info()
    nw = info.num_cores * info.num_subcores
    flat = token_ids.reshape(nw, (b * h) // (nw * _CHUNK), _CHUNK).astype(jnp.int32)
    emb = _make_gather(b * h, v, d)
    out = emb(flat, embedding_matrix)
    return out.reshape(b, h, d)
